# baseline (speedup 1.0000x reference)
import functools
import math

import jax
import jax.numpy as jnp
from jax import lax
from jax.experimental import pallas as pl
from jax.experimental.pallas import tpu as pltpu
from jax.experimental.pallas import tpu_sc as plsc

B, S, D = 1, 2048, 2048
H, KV, HD = 16, 4, 128
E, TOPK, FF = 8, 2, 768
EPS = 1e-6
THETA = 10000.0
NEG = -1e9

NA = S * TOPK
BLK = 128
NB = NA // BLK + E
P = NB * BLK

f32 = jnp.float32
bf16 = jnp.bfloat16


def _qkv_body(h_ref, lnw_ref, qw_ref, kw_ref, vw_ref, q_ref, k_ref, v_ref):
    x = h_ref[...]
    ms = jnp.mean(x * x, axis=1, keepdims=True)
    xn = x * jax.lax.rsqrt(ms + EPS) * lnw_ref[...]
    xb = xn.astype(bf16)
    q_ref[...] = jnp.dot(xb, qw_ref[...], preferred_element_type=f32)
    k_ref[...] = jnp.dot(xb, kw_ref[...], preferred_element_type=f32)
    v_ref[...] = jnp.dot(xb, vw_ref[...],
                         preferred_element_type=f32).astype(bf16)


def _qkv(h2d, input_ln_w, q_w, k_w, v_w, sb=256):
    n = S // sb
    return pl.pallas_call(
        _qkv_body,
        grid=(n,),
        in_specs=[
            pl.BlockSpec((sb, D), lambda i: (i, 0)),
            pl.BlockSpec((1, D), lambda i: (0, 0)),
            pl.BlockSpec((D, H * HD), lambda i: (0, 0)),
            pl.BlockSpec((D, KV * HD), lambda i: (0, 0)),
            pl.BlockSpec((D, KV * HD), lambda i: (0, 0)),
        ],
        out_specs=[
            pl.BlockSpec((sb, H * HD), lambda i: (i, 0)),
            pl.BlockSpec((sb, KV * HD), lambda i: (i, 0)),
            pl.BlockSpec((sb, KV * HD), lambda i: (i, 0)),
        ],
        out_shape=[
            jax.ShapeDtypeStruct((S, H * HD), f32),
            jax.ShapeDtypeStruct((S, KV * HD), f32),
            jax.ShapeDtypeStruct((S, KV * HD), bf16),
        ],
    )(h2d, input_ln_w.reshape(1, D), q_w.astype(bf16), k_w.astype(bf16),
      v_w.astype(bf16))


def _rot_cat(x):
    return jnp.concatenate([-x[:, HD // 2:], x[:, :HD // 2]], axis=1)


def _krope_body(k_ref, cos_ref, sin_ref, lnw_ref, o_ref):
    k = k_ref[...]
    ms = jnp.mean(k * k, axis=1, keepdims=True)
    kn = k * jax.lax.rsqrt(ms + EPS) * lnw_ref[...]
    o_ref[...] = (kn * cos_ref[...] + _rot_cat(kn) * sin_ref[...]).astype(bf16)


def _krope(k2d, cos_t, sin_t, k_ln_w, sb=512):
    n = S // sb
    return pl.pallas_call(
        _krope_body,
        grid=(KV, n),
        in_specs=[
            pl.BlockSpec((sb, HD), lambda kv, i: (i, kv)),
            pl.BlockSpec((sb, HD), lambda kv, i: (i, 0)),
            pl.BlockSpec((sb, HD), lambda kv, i: (i, 0)),
            pl.BlockSpec((1, HD), lambda kv, i: (0, 0)),
        ],
        out_specs=pl.BlockSpec((sb, HD), lambda kv, i: (i, kv)),
        out_shape=jax.ShapeDtypeStruct((S, KV * HD), bf16),
    )(k2d, cos_t, sin_t, k_ln_w.reshape(1, HD))


def _attn_body(q_ref, k_ref, v_ref, cos_ref, sin_ref, lnw_ref, o_ref, sc_ref,
               *, qb, kb):
    i = pl.program_id(1)
    q = q_ref[...]
    ms = jnp.mean(q * q, axis=1, keepdims=True)
    qn = q * jax.lax.rsqrt(ms + EPS) * lnw_ref[...]
    qr = (qn * cos_ref[...] + _rot_cat(qn) * sin_ref[...]).astype(bf16)
    nj = i * qb // kb + 1
    row = i * qb + jax.lax.broadcasted_iota(jnp.int32, (qb, kb), 0)
    sc_ref[...] = jnp.full((qb, S), NEG, f32)

    def sbody(j, carry):
        kblk = k_ref[pl.ds(j * kb, kb), :]
        s = jax.lax.dot_general(
            qr, kblk, (((1,), (1,)), ((), ())),
            preferred_element_type=f32) * (1.0 / math.sqrt(HD))
        col = j * kb + jax.lax.broadcasted_iota(jnp.int32, (qb, kb), 1)
        sc_ref[:, pl.ds(j * kb, kb)] = jnp.where(col <= row, s, NEG)
        return carry

    jax.lax.fori_loop(0, nj, sbody, 0)
    sc = sc_ref[...]
    m = jnp.max(sc, axis=1, keepdims=True)
    p = jnp.exp(sc - m)
    sc_ref[...] = p / jnp.sum(p, axis=1, keepdims=True)

    def pbody(j, acc):
        a = sc_ref[:, pl.ds(j * kb, kb)].astype(bf16)
        vblk = v_ref[pl.ds(j * kb, kb), :]
        return acc + jnp.dot(a, vblk, preferred_element_type=f32)

    acc = jax.lax.fori_loop(0, nj, pbody, jnp.zeros((qb, HD), f32))
    o_ref[...] = acc.astype(bf16)


def _attention(q2d, kr2d, v2d, cos_t, sin_t, q_ln_w, qb=256, kb=512):
    n = S // qb
    return pl.pallas_call(
        functools.partial(_attn_body, qb=qb, kb=kb),
        grid=(H, n),
        in_specs=[
            pl.BlockSpec((qb, HD), lambda h, i: (i, h)),
            pl.BlockSpec((S, HD), lambda h, i: (0, h // (H // KV))),
            pl.BlockSpec((S, HD), lambda h, i: (0, h // (H // KV))),
            pl.BlockSpec((qb, HD), lambda h, i: (i, 0)),
            pl.BlockSpec((qb, HD), lambda h, i: (i, 0)),
            pl.BlockSpec((1, HD), lambda h, i: (0, 0)),
        ],
        out_specs=pl.BlockSpec((qb, HD), lambda h, i: (i, h)),
        out_shape=jax.ShapeDtypeStruct((S, H * HD), bf16),
        scratch_shapes=[pltpu.VMEM((qb, S), f32)],
    )(q2d, kr2d, v2d, cos_t, sin_t, q_ln_w.reshape(1, HD))


def _oproj_body(o_ref, ow_ref, hid_ref, lnw_ref, rw_ref, hs_ref, xn_ref,
                lg_ref):
    att = jnp.dot(o_ref[...], ow_ref[...], preferred_element_type=f32)
    hs = hid_ref[...] + att
    hs_ref[...] = hs
    ms = jnp.mean(hs * hs, axis=1, keepdims=True)
    xn = hs * jax.lax.rsqrt(ms + EPS) * lnw_ref[...]
    xn_ref[...] = xn.reshape(xn.shape[0], D // 128, 128)
    lg_ref[...] = jnp.dot(xn.astype(bf16), rw_ref[...],
                          preferred_element_type=f32)


def _oproj(o2d, o_w, hid2d, post_ln_w, rw_pad, sb=256):
    n = S // sb
    return pl.pallas_call(
        _oproj_body,
        grid=(n,),
        in_specs=[
            pl.BlockSpec((sb, H * HD), lambda i: (i, 0)),
            pl.BlockSpec((H * HD, D), lambda i: (0, 0)),
            pl.BlockSpec((sb, D), lambda i: (i, 0)),
            pl.BlockSpec((1, D), lambda i: (0, 0)),
            pl.BlockSpec((D, 128), lambda i: (0, 0)),
        ],
        out_specs=[
            pl.BlockSpec((sb, D), lambda i: (i, 0)),
            pl.BlockSpec((sb, D // 128, 128), lambda i: (i, 0, 0)),
            pl.BlockSpec((sb, 128), lambda i: (i, 0)),
        ],
        out_shape=[
            jax.ShapeDtypeStruct((S, D), f32),
            jax.ShapeDtypeStruct((S, D // 128, 128), f32),
            jax.ShapeDtypeStruct((S, 128), f32),
        ],
    )(o2d, o_w.astype(bf16), hid2d, post_ln_w.reshape(1, D), rw_pad)


AB = 512


def _route_body(lg_ref, ej_ref, wj_ref, rank_ref, cnt_ref, cnt):
    j = pl.program_id(0)
    tb = pl.program_id(1)

    @pl.when((j == 0) & (tb == 0))
    def _():
        cnt[...] = jnp.zeros_like(cnt)

    l = lg_ref[...]
    lane = jax.lax.broadcasted_iota(jnp.int32, (AB, 128), 1)
    valid = lane < E
    l = jnp.where(valid, l, -1e30)
    m0 = jnp.max(l, axis=1, keepdims=True)
    i0 = jnp.min(jnp.where(l >= m0, lane, 1000), axis=1, keepdims=True)
    l1 = jnp.where(lane == i0, -1e30, l)
    m1 = jnp.max(l1, axis=1, keepdims=True)
    i1 = jnp.min(jnp.where(l1 >= m1, lane, 1000), axis=1, keepdims=True)
    w0 = 1.0 / (1.0 + jnp.exp(m1 - m0))
    ej = jnp.where(j == 0, i0, i1)
    wj = jnp.where(j == 0, w0, 1.0 - w0)
    onehot = (lane == ej).astype(f32)
    rowi = jax.lax.broadcasted_iota(jnp.int32, (AB, AB), 0)
    coli = jax.lax.broadcasted_iota(jnp.int32, (AB, AB), 1)
    ltri = (coli < rowi).astype(bf16)
    pref = jnp.dot(ltri, onehot.astype(bf16), preferred_element_type=f32)
    rank = jnp.sum(onehot * (pref + cnt[...]), axis=1, keepdims=True)
    ej_ref[...] = ej.astype(f32)
    wj_ref[...] = wj
    rank_ref[...] = rank
    cnt[...] += jnp.sum(onehot, axis=0, keepdims=True)
    cnt_ref[...] = cnt[...]


def _route_sort(logits):
    nt = S // AB
    return pl.pallas_call(
        _route_body,
        grid=(TOPK, nt),
        in_specs=[pl.BlockSpec((AB, 128), lambda j, tb: (tb, 0))],
        out_specs=[
            pl.BlockSpec((AB, 1), lambda j, tb: (j * nt + tb, 0)),
            pl.BlockSpec((AB, 1), lambda j, tb: (j * nt + tb, 0)),
            pl.BlockSpec((AB, 1), lambda j, tb: (j * nt + tb, 0)),
            pl.BlockSpec((1, 128), lambda j, tb: (0, 0)),
        ],
        out_shape=[
            jax.ShapeDtypeStruct((NA, 1), f32),
            jax.ShapeDtypeStruct((NA, 1), f32),
            jax.ShapeDtypeStruct((NA, 1), f32),
            jax.ShapeDtypeStruct((1, 128), f32),
        ],
        scratch_shapes=[pltpu.VMEM((1, 128), f32)],
    )(logits)


def _desc_body(cnt_ref, ej_ref, rank_ref, ppos_ref, rows_ref, bexp_ref):
    cnts = [cnt_ref[0, e] for e in range(E)]
    pcs = [jnp.ceil(c / BLK) * BLK for c in cnts]
    pb = [f32(0.0)]
    ab = [f32(0.0)]
    for e in range(E):
        pb.append(pb[e] + pcs[e])
        ab.append(ab[e] + cnts[e])
    ej = ej_ref[...]
    acc = jnp.zeros_like(ej)
    for e in range(E):
        acc += jnp.where(ej == e, pb[e], 0.0)
    ppos = acc + rank_ref[...]
    ppos_ref[...] = ppos
    ascan = 512
    rows = jnp.zeros((1, P), f32)
    for blk in range(NA // ascan):
        ppos_blk = ppos[blk * ascan:(blk + 1) * ascan]
        piota = jax.lax.broadcasted_iota(jnp.int32, (ascan, P), 1).astype(f32)
        m1 = (ppos_blk == piota).astype(f32)
        lane = jax.lax.broadcasted_iota(jnp.int32, (1, ascan), 1)
        tok = ((blk * ascan + lane) % S).astype(f32)
        rows += jax.lax.dot_general(
            tok, m1, (((1,), (0,)), ((), ())),
            precision=jax.lax.Precision.HIGHEST,
            preferred_element_type=f32)
    rows_ref[...] = rows
    lb = jax.lax.broadcasted_iota(jnp.int32, (1, 128), 1).astype(f32) * BLK
    bacc = -jnp.ones((1, 128), f32)
    for e in range(E):
        bacc += jnp.where(lb >= pb[e], 1.0, 0.0)
    bexp_ref[...] = bacc


def _desc(cnt, ej, rank):
    return pl.pallas_call(
        _desc_body,
        grid=(1,),
        in_specs=[
            pl.BlockSpec(memory_space=pltpu.SMEM),
            pl.BlockSpec((NA, 1), lambda i: (0, 0)),
            pl.BlockSpec((NA, 1), lambda i: (0, 0)),
        ],
        out_specs=[
            pl.BlockSpec((NA, 1), lambda i: (0, 0)),
            pl.BlockSpec((1, P), lambda i: (0, 0)),
            pl.BlockSpec((1, 128), lambda i: (0, 0)),
        ],
        out_shape=[
            jax.ShapeDtypeStruct((NA, 1), f32),
            jax.ShapeDtypeStruct((1, P), f32),
            jax.ShapeDtypeStruct((1, 128), f32),
        ],
    )(cnt, ej, rank)


def _make_sc_gather(nrows_tab, nrows_out, sl, dtype):
    info = plsc.get_sparse_core_info()
    nw = info.num_cores * info.num_subcores
    n_per_w = nrows_out // nw
    row_bytes = sl * 128 * jnp.dtype(dtype).itemsize
    ch = 8
    cand = 8
    while cand <= n_per_w:
        if n_per_w % cand == 0 and 2 * cand * row_bytes <= 440_000:
            ch = cand
        cand += 8
    nch = n_per_w // ch
    mesh = plsc.VectorSubcoreMesh(core_axis_name="c", subcore_axis_name="s")

    @functools.partial(
        pl.kernel, mesh=mesh,
        out_type=jax.ShapeDtypeStruct((nrows_out, sl, 128), dtype),
        scratch_types=[
            pltpu.VMEM((2, ch), jnp.int32),
            pltpu.VMEM((ch, sl, 128), dtype),
            pltpu.VMEM((ch, sl, 128), dtype),
            pltpu.SemaphoreType.DMA,
            pltpu.SemaphoreType.DMA,
            pltpu.SemaphoreType.DMA,
            pltpu.SemaphoreType.DMA,
        ],
    )
    def k(tab_hbm, idx_hbm, out_hbm, idx_v, buf0, buf1, g0, g1, w0, w1):
        wid = lax.axis_index("s") * info.num_cores + lax.axis_index("c")
        base = wid * n_per_w
        bufs = (buf0, buf1)
        gs = (g0, g1)
        ws = (w0, w1)

        def start_gather(c):
            b = c & 1
            pltpu.sync_copy(idx_hbm.at[pl.ds(base + c * ch, ch)],
                            idx_v.at[b])
            return pltpu.async_copy(tab_hbm.at[idx_v.at[b]], bufs[b], gs[b])

        gh = {0: start_gather(0)}
        wh = {}
        for c in range(nch):
            b = c & 1
            gh[c].wait()
            if c + 1 < nch:
                if c >= 1:
                    wh[c - 1].wait()
                gh[c + 1] = start_gather(c + 1)
            wh[c] = pltpu.async_copy(
                bufs[b], out_hbm.at[pl.ds(base + c * ch, ch)], ws[b])
        if nch >= 2:
            wh[nch - 2].wait()
        wh[nch - 1].wait()

    return k


def _moe_body(be_ref, x_ref, gw_ref, uw_ref, dw_ref, y_ref, gwb, uwb, dwb):
    b = pl.program_id(0)
    prev = be_ref[jnp.maximum(b - 1, 0)]

    @pl.when((b == 0) | (be_ref[b] != prev))
    def _():
        gwb[...] = gw_ref[0].astype(bf16)
        uwb[...] = uw_ref[0].astype(bf16)
        dwb[...] = dw_ref[0].astype(bf16)

    x = x_ref[...].reshape(BLK, D).astype(bf16)
    g = jnp.dot(x, gwb[...], preferred_element_type=f32)
    u = jnp.dot(x, uwb[...], preferred_element_type=f32)
    a = ((g / (1.0 + jnp.exp(-g))) * u).astype(bf16)
    y = jnp.dot(a, dwb[...], preferred_element_type=f32)
    y_ref[...] = y.reshape(BLK, D // 128, 128)


def _moe_grouped(xdisp3, gate_w, up_w, down_w, bexp):
    grid_spec = pltpu.PrefetchScalarGridSpec(
        num_scalar_prefetch=1,
        grid=(NB,),
        in_specs=[
            pl.BlockSpec((BLK, D // 128, 128), lambda b, be: (b, 0, 0)),
            pl.BlockSpec((1, D, FF), lambda b, be: (be[b], 0, 0)),
            pl.BlockSpec((1, D, FF), lambda b, be: (be[b], 0, 0)),
            pl.BlockSpec((1, FF, D), lambda b, be: (be[b], 0, 0)),
        ],
        out_specs=pl.BlockSpec((BLK, D // 128, 128), lambda b, be: (b, 0, 0)),
        scratch_shapes=[
            pltpu.VMEM((D, FF), bf16),
            pltpu.VMEM((D, FF), bf16),
            pltpu.VMEM((FF, D), bf16),
        ],
    )
    return pl.pallas_call(
        _moe_body,
        grid_spec=grid_spec,
        out_shape=jax.ShapeDtypeStruct((P, D // 128, 128), f32),
    )(bexp, xdisp3, gate_w, up_w, down_w)


def _comb_body(hs_ref, y0_ref, y1_ref, w0_ref, w1_ref, out_ref):
    sb = hs_ref.shape[0]
    y0 = y0_ref[...].reshape(sb, D)
    y1 = y1_ref[...].reshape(sb, D)
    out_ref[...] = (hs_ref[...] + w0_ref[...] * y0 + w1_ref[...] * y1)


def _combine(hs2d, yg, w0, w1, sb=256):
    n = S // sb
    return pl.pallas_call(
        _comb_body,
        grid=(n,),
        in_specs=[
            pl.BlockSpec((sb, D), lambda i: (i, 0)),
            pl.BlockSpec((sb, D // 128, 128), lambda i: (i, 0, 0)),
            pl.BlockSpec((sb, D // 128, 128), lambda i: (i + S // sb, 0, 0)),
            pl.BlockSpec((sb, 1), lambda i: (i, 0)),
            pl.BlockSpec((sb, 1), lambda i: (i, 0)),
        ],
        out_specs=pl.BlockSpec((sb, D), lambda i: (i, 0)),
        out_shape=jax.ShapeDtypeStruct((S, D), f32),
    )(hs2d, yg, yg, w0, w1)


def kernel(hidden_states, position_ids, input_ln_w, q_w, k_w, v_w, o_w,
           q_ln_w, k_ln_w, post_ln_w, router_w, gate_w, up_w, down_w):
    h2d = hidden_states.reshape(S, D)
    pos = position_ids.reshape(S).astype(f32)
    inv = 1.0 / (THETA ** (jnp.arange(0, HD, 2, dtype=f32) / HD))
    ang = pos[:, None] * inv[None, :]
    cos_t = jnp.concatenate([jnp.cos(ang), jnp.cos(ang)], axis=1)
    sin_t = jnp.concatenate([jnp.sin(ang), jnp.sin(ang)], axis=1)

    q2d, k2d, v2d = _qkv(h2d, input_ln_w, q_w, k_w, v_w)
    kr2d = _krope(k2d, cos_t, sin_t, k_ln_w)
    o2d = _attention(q2d, kr2d, v2d, cos_t, sin_t, q_ln_w)

    rw_pad = jnp.pad(router_w, ((0, 0), (0, 128 - E))).astype(bf16)
    hs2d, xn3, logits = _oproj(o2d, o_w, h2d, post_ln_w, rw_pad)

    ej, wj, rank, cnt = _route_sort(logits)
    ppos, rows, bexp_l = _desc(cnt, ej, rank)

    rows_i = rows.reshape(P).astype(jnp.int32)
    ppos_i = ppos.reshape(NA).astype(jnp.int32)
    bexp = bexp_l.reshape(128)[:NB].astype(jnp.int32)
    w0 = wj.reshape(TOPK, S, 1)[0]
    w1 = wj.reshape(TOPK, S, 1)[1]

    nl = D // 128
    xdisp3 = _make_sc_gather(S, P, nl, f32)(xn3, rows_i)
    ydisp3 = _moe_grouped(xdisp3, gate_w, up_w, down_w, bexp)
    yg = _make_sc_gather(P, NA, nl, f32)(ydisp3, ppos_i)
    out = _combine(hs2d, yg, w0, w1)
    return out.reshape(B, S, D)


# speedup vs baseline: 1.1025x; 1.1025x over previous
import functools
import math

import jax
import jax.numpy as jnp
from jax import lax
from jax.experimental import pallas as pl
from jax.experimental.pallas import tpu as pltpu
from jax.experimental.pallas import tpu_sc as plsc

B, S, D = 1, 2048, 2048
H, KV, HD = 16, 4, 128
E, TOPK, FF = 8, 2, 768
EPS = 1e-6
THETA = 10000.0
NEG = -1e9

NA = S * TOPK
BLK = 128
NB = NA // BLK + E
P = NB * BLK

f32 = jnp.float32
bf16 = jnp.bfloat16


def _qkv_body(h_ref, lnw_ref, qw_ref, kw_ref, vw_ref, q_ref, k_ref, v_ref):
    x = h_ref[...]
    ms = jnp.mean(x * x, axis=1, keepdims=True)
    xn = x * jax.lax.rsqrt(ms + EPS) * lnw_ref[...]
    xb = xn.astype(bf16)
    q_ref[...] = jnp.dot(xb, qw_ref[...], preferred_element_type=f32)
    k_ref[...] = jnp.dot(xb, kw_ref[...], preferred_element_type=f32)
    v_ref[...] = jnp.dot(xb, vw_ref[...],
                         preferred_element_type=f32).astype(bf16)


def _qkv(h2d, input_ln_w, q_w, k_w, v_w, sb=256):
    n = S // sb
    return pl.pallas_call(
        _qkv_body,
        grid=(n,),
        in_specs=[
            pl.BlockSpec((sb, D), lambda i: (i, 0)),
            pl.BlockSpec((1, D), lambda i: (0, 0)),
            pl.BlockSpec((D, H * HD), lambda i: (0, 0)),
            pl.BlockSpec((D, KV * HD), lambda i: (0, 0)),
            pl.BlockSpec((D, KV * HD), lambda i: (0, 0)),
        ],
        out_specs=[
            pl.BlockSpec((sb, H * HD), lambda i: (i, 0)),
            pl.BlockSpec((sb, KV * HD), lambda i: (i, 0)),
            pl.BlockSpec((sb, KV * HD), lambda i: (i, 0)),
        ],
        out_shape=[
            jax.ShapeDtypeStruct((S, H * HD), f32),
            jax.ShapeDtypeStruct((S, KV * HD), f32),
            jax.ShapeDtypeStruct((S, KV * HD), bf16),
        ],
    )(h2d, input_ln_w.reshape(1, D), q_w.astype(bf16), k_w.astype(bf16),
      v_w.astype(bf16))


def _rot_cat(x):
    return jnp.concatenate([-x[:, HD // 2:], x[:, :HD // 2]], axis=1)


def _krope_body(k_ref, cos_ref, sin_ref, lnw_ref, o_ref):
    k = k_ref[...]
    ms = jnp.mean(k * k, axis=1, keepdims=True)
    kn = k * jax.lax.rsqrt(ms + EPS) * lnw_ref[...]
    o_ref[...] = (kn * cos_ref[...] + _rot_cat(kn) * sin_ref[...]).astype(bf16)


def _krope(k2d, cos_t, sin_t, k_ln_w, sb=512):
    n = S // sb
    return pl.pallas_call(
        _krope_body,
        grid=(KV, n),
        in_specs=[
            pl.BlockSpec((sb, HD), lambda kv, i: (i, kv)),
            pl.BlockSpec((sb, HD), lambda kv, i: (i, 0)),
            pl.BlockSpec((sb, HD), lambda kv, i: (i, 0)),
            pl.BlockSpec((1, HD), lambda kv, i: (0, 0)),
        ],
        out_specs=pl.BlockSpec((sb, HD), lambda kv, i: (i, kv)),
        out_shape=jax.ShapeDtypeStruct((S, KV * HD), bf16),
    )(k2d, cos_t, sin_t, k_ln_w.reshape(1, HD))


def _attn_body(q_ref, k_ref, v_ref, cos_ref, sin_ref, lnw_ref, o_ref, sc_ref,
               *, qb, kb):
    i = pl.program_id(1)
    q = q_ref[...]
    ms = jnp.mean(q * q, axis=1, keepdims=True)
    qn = q * jax.lax.rsqrt(ms + EPS) * lnw_ref[...]
    qr = (qn * cos_ref[...] + _rot_cat(qn) * sin_ref[...]).astype(bf16)
    del sc_ref, kb
    scores = jax.lax.dot_general(
        qr, k_ref[...], (((1,), (1,)), ((), ())),
        preferred_element_type=f32) * (1.0 / math.sqrt(HD))
    row = i * qb + jax.lax.broadcasted_iota(jnp.int32, (qb, S), 0)
    col = jax.lax.broadcasted_iota(jnp.int32, (qb, S), 1)
    scores = jnp.where(col <= row, scores, NEG)
    m = jnp.max(scores, axis=1, keepdims=True)
    p = jnp.exp(scores - m)
    attn = (p / jnp.sum(p, axis=1, keepdims=True)).astype(bf16)
    o_ref[...] = jnp.dot(attn, v_ref[...],
                         preferred_element_type=f32).astype(bf16)


def _attention(q2d, kr2d, v2d, cos_t, sin_t, q_ln_w, qb=256, kb=512):
    n = S // qb
    return pl.pallas_call(
        functools.partial(_attn_body, qb=qb, kb=kb),
        grid=(H, n),
        in_specs=[
            pl.BlockSpec((qb, HD), lambda h, i: (i, h)),
            pl.BlockSpec((S, HD), lambda h, i: (0, h // (H // KV))),
            pl.BlockSpec((S, HD), lambda h, i: (0, h // (H // KV))),
            pl.BlockSpec((qb, HD), lambda h, i: (i, 0)),
            pl.BlockSpec((qb, HD), lambda h, i: (i, 0)),
            pl.BlockSpec((1, HD), lambda h, i: (0, 0)),
        ],
        out_specs=pl.BlockSpec((qb, HD), lambda h, i: (i, h)),
        out_shape=jax.ShapeDtypeStruct((S, H * HD), bf16),
        scratch_shapes=[pltpu.VMEM((qb, S), f32)],
    )(q2d, kr2d, v2d, cos_t, sin_t, q_ln_w.reshape(1, HD))


def _oproj_body(o_ref, ow_ref, hid_ref, lnw_ref, rw_ref, hs_ref, xn_ref,
                lg_ref):
    att = jnp.dot(o_ref[...], ow_ref[...], preferred_element_type=f32)
    hs = hid_ref[...] + att
    hs_ref[...] = hs
    ms = jnp.mean(hs * hs, axis=1, keepdims=True)
    xn = hs * jax.lax.rsqrt(ms + EPS) * lnw_ref[...]
    xn_ref[...] = xn.reshape(xn.shape[0], D // 128, 128)
    lg_ref[...] = jnp.dot(xn.astype(bf16), rw_ref[...],
                          preferred_element_type=f32)


def _oproj(o2d, o_w, hid2d, post_ln_w, rw_pad, sb=256):
    n = S // sb
    return pl.pallas_call(
        _oproj_body,
        grid=(n,),
        in_specs=[
            pl.BlockSpec((sb, H * HD), lambda i: (i, 0)),
            pl.BlockSpec((H * HD, D), lambda i: (0, 0)),
            pl.BlockSpec((sb, D), lambda i: (i, 0)),
            pl.BlockSpec((1, D), lambda i: (0, 0)),
            pl.BlockSpec((D, 128), lambda i: (0, 0)),
        ],
        out_specs=[
            pl.BlockSpec((sb, D), lambda i: (i, 0)),
            pl.BlockSpec((sb, D // 128, 128), lambda i: (i, 0, 0)),
            pl.BlockSpec((sb, 128), lambda i: (i, 0)),
        ],
        out_shape=[
            jax.ShapeDtypeStruct((S, D), f32),
            jax.ShapeDtypeStruct((S, D // 128, 128), f32),
            jax.ShapeDtypeStruct((S, 128), f32),
        ],
    )(o2d, o_w.astype(bf16), hid2d, post_ln_w.reshape(1, D), rw_pad)


AB = 512


def _route_body(lg_ref, ej_ref, wj_ref, rank_ref, cnt_ref, cnt):
    j = pl.program_id(0)
    tb = pl.program_id(1)

    @pl.when((j == 0) & (tb == 0))
    def _():
        cnt[...] = jnp.zeros_like(cnt)

    l = lg_ref[...]
    lane = jax.lax.broadcasted_iota(jnp.int32, (AB, 128), 1)
    valid = lane < E
    l = jnp.where(valid, l, -1e30)
    m0 = jnp.max(l, axis=1, keepdims=True)
    i0 = jnp.min(jnp.where(l >= m0, lane, 1000), axis=1, keepdims=True)
    l1 = jnp.where(lane == i0, -1e30, l)
    m1 = jnp.max(l1, axis=1, keepdims=True)
    i1 = jnp.min(jnp.where(l1 >= m1, lane, 1000), axis=1, keepdims=True)
    w0 = 1.0 / (1.0 + jnp.exp(m1 - m0))
    ej = jnp.where(j == 0, i0, i1)
    wj = jnp.where(j == 0, w0, 1.0 - w0)
    onehot = (lane == ej).astype(f32)
    rowi = jax.lax.broadcasted_iota(jnp.int32, (AB, AB), 0)
    coli = jax.lax.broadcasted_iota(jnp.int32, (AB, AB), 1)
    ltri = (coli < rowi).astype(bf16)
    pref = jnp.dot(ltri, onehot.astype(bf16), preferred_element_type=f32)
    rank = jnp.sum(onehot * (pref + cnt[...]), axis=1, keepdims=True)
    ej_ref[...] = ej.astype(f32)
    wj_ref[...] = wj
    rank_ref[...] = rank
    cnt[...] += jnp.sum(onehot, axis=0, keepdims=True)
    cnt_ref[...] = cnt[...]


def _route_sort(logits):
    nt = S // AB
    return pl.pallas_call(
        _route_body,
        grid=(TOPK, nt),
        in_specs=[pl.BlockSpec((AB, 128), lambda j, tb: (tb, 0))],
        out_specs=[
            pl.BlockSpec((AB, 1), lambda j, tb: (j * nt + tb, 0)),
            pl.BlockSpec((AB, 1), lambda j, tb: (j * nt + tb, 0)),
            pl.BlockSpec((AB, 1), lambda j, tb: (j * nt + tb, 0)),
            pl.BlockSpec((1, 128), lambda j, tb: (0, 0)),
        ],
        out_shape=[
            jax.ShapeDtypeStruct((NA, 1), f32),
            jax.ShapeDtypeStruct((NA, 1), f32),
            jax.ShapeDtypeStruct((NA, 1), f32),
            jax.ShapeDtypeStruct((1, 128), f32),
        ],
        scratch_shapes=[pltpu.VMEM((1, 128), f32)],
    )(logits)


def _desc_body(cnt_ref, ej_ref, rank_ref, ppos_ref, rows_ref, bexp_ref):
    cnts = [cnt_ref[0, e] for e in range(E)]
    pcs = [jnp.ceil(c / BLK) * BLK for c in cnts]
    pb = [f32(0.0)]
    ab = [f32(0.0)]
    for e in range(E):
        pb.append(pb[e] + pcs[e])
        ab.append(ab[e] + cnts[e])
    ej = ej_ref[...]
    acc = jnp.zeros_like(ej)
    for e in range(E):
        acc += jnp.where(ej == e, pb[e], 0.0)
    ppos = acc + rank_ref[...]
    ppos_ref[...] = ppos
    ascan = 512
    rows = jnp.zeros((1, P), f32)
    for blk in range(NA // ascan):
        ppos_blk = ppos[blk * ascan:(blk + 1) * ascan]
        piota = jax.lax.broadcasted_iota(jnp.int32, (ascan, P), 1).astype(f32)
        m1 = (ppos_blk == piota).astype(f32)
        lane = jax.lax.broadcasted_iota(jnp.int32, (1, ascan), 1)
        tok = ((blk * ascan + lane) % S).astype(f32)
        rows += jax.lax.dot_general(
            tok, m1, (((1,), (0,)), ((), ())),
            precision=jax.lax.Precision.HIGHEST,
            preferred_element_type=f32)
    rows_ref[...] = rows
    lb = jax.lax.broadcasted_iota(jnp.int32, (1, 128), 1).astype(f32) * BLK
    bacc = -jnp.ones((1, 128), f32)
    for e in range(E):
        bacc += jnp.where(lb >= pb[e], 1.0, 0.0)
    bexp_ref[...] = bacc


def _desc(cnt, ej, rank):
    return pl.pallas_call(
        _desc_body,
        grid=(1,),
        in_specs=[
            pl.BlockSpec(memory_space=pltpu.SMEM),
            pl.BlockSpec((NA, 1), lambda i: (0, 0)),
            pl.BlockSpec((NA, 1), lambda i: (0, 0)),
        ],
        out_specs=[
            pl.BlockSpec((NA, 1), lambda i: (0, 0)),
            pl.BlockSpec((1, P), lambda i: (0, 0)),
            pl.BlockSpec((1, 128), lambda i: (0, 0)),
        ],
        out_shape=[
            jax.ShapeDtypeStruct((NA, 1), f32),
            jax.ShapeDtypeStruct((1, P), f32),
            jax.ShapeDtypeStruct((1, 128), f32),
        ],
    )(cnt, ej, rank)


def _make_sc_gather(nrows_tab, nrows_out, sl, dtype):
    info = plsc.get_sparse_core_info()
    nw = info.num_cores * info.num_subcores
    n_per_w = nrows_out // nw
    row_bytes = sl * 128 * jnp.dtype(dtype).itemsize
    ch = 8
    cand = 8
    while cand <= n_per_w:
        if n_per_w % cand == 0 and 2 * cand * row_bytes <= 440_000:
            ch = cand
        cand += 8
    nch = n_per_w // ch
    mesh = plsc.VectorSubcoreMesh(core_axis_name="c", subcore_axis_name="s")

    @functools.partial(
        pl.kernel, mesh=mesh,
        out_type=jax.ShapeDtypeStruct((nrows_out, sl, 128), dtype),
        scratch_types=[
            pltpu.VMEM((2, ch), jnp.int32),
            pltpu.VMEM((ch, sl, 128), dtype),
            pltpu.VMEM((ch, sl, 128), dtype),
            pltpu.SemaphoreType.DMA,
            pltpu.SemaphoreType.DMA,
            pltpu.SemaphoreType.DMA,
            pltpu.SemaphoreType.DMA,
        ],
    )
    def k(tab_hbm, idx_hbm, out_hbm, idx_v, buf0, buf1, g0, g1, w0, w1):
        wid = lax.axis_index("s") * info.num_cores + lax.axis_index("c")
        base = wid * n_per_w
        bufs = (buf0, buf1)
        gs = (g0, g1)
        ws = (w0, w1)

        def start_gather(c):
            b = c & 1
            pltpu.sync_copy(idx_hbm.at[pl.ds(base + c * ch, ch)],
                            idx_v.at[b])
            return pltpu.async_copy(tab_hbm.at[idx_v.at[b]], bufs[b], gs[b])

        gh = {0: start_gather(0)}
        wh = {}
        for c in range(nch):
            b = c & 1
            gh[c].wait()
            if c + 1 < nch:
                if c >= 1:
                    wh[c - 1].wait()
                gh[c + 1] = start_gather(c + 1)
            wh[c] = pltpu.async_copy(
                bufs[b], out_hbm.at[pl.ds(base + c * ch, ch)], ws[b])
        if nch >= 2:
            wh[nch - 2].wait()
        wh[nch - 1].wait()

    return k


def _moe_body(be_ref, x_ref, gw_ref, uw_ref, dw_ref, y_ref, gwb, uwb, dwb):
    b = pl.program_id(0)
    prev = be_ref[jnp.maximum(b - 1, 0)]

    @pl.when((b == 0) | (be_ref[b] != prev))
    def _():
        gwb[...] = gw_ref[0].astype(bf16)
        uwb[...] = uw_ref[0].astype(bf16)
        dwb[...] = dw_ref[0].astype(bf16)

    x = x_ref[...].reshape(BLK, D).astype(bf16)
    g = jnp.dot(x, gwb[...], preferred_element_type=f32)
    u = jnp.dot(x, uwb[...], preferred_element_type=f32)
    a = ((g / (1.0 + jnp.exp(-g))) * u).astype(bf16)
    y = jnp.dot(a, dwb[...], preferred_element_type=f32)
    y_ref[...] = y.reshape(BLK, D // 128, 128)


def _moe_grouped(xdisp3, gate_w, up_w, down_w, bexp):
    grid_spec = pltpu.PrefetchScalarGridSpec(
        num_scalar_prefetch=1,
        grid=(NB,),
        in_specs=[
            pl.BlockSpec((BLK, D // 128, 128), lambda b, be: (b, 0, 0)),
            pl.BlockSpec((1, D, FF), lambda b, be: (be[b], 0, 0)),
            pl.BlockSpec((1, D, FF), lambda b, be: (be[b], 0, 0)),
            pl.BlockSpec((1, FF, D), lambda b, be: (be[b], 0, 0)),
        ],
        out_specs=pl.BlockSpec((BLK, D // 128, 128), lambda b, be: (b, 0, 0)),
        scratch_shapes=[
            pltpu.VMEM((D, FF), bf16),
            pltpu.VMEM((D, FF), bf16),
            pltpu.VMEM((FF, D), bf16),
        ],
    )
    return pl.pallas_call(
        _moe_body,
        grid_spec=grid_spec,
        out_shape=jax.ShapeDtypeStruct((P, D // 128, 128), f32),
    )(bexp, xdisp3, gate_w, up_w, down_w)


def _comb_body(hs_ref, y0_ref, y1_ref, w0_ref, w1_ref, out_ref):
    sb = hs_ref.shape[0]
    y0 = y0_ref[...].reshape(sb, D)
    y1 = y1_ref[...].reshape(sb, D)
    out_ref[...] = (hs_ref[...] + w0_ref[...] * y0 + w1_ref[...] * y1)


def _combine(hs2d, yg, w0, w1, sb=256):
    n = S // sb
    return pl.pallas_call(
        _comb_body,
        grid=(n,),
        in_specs=[
            pl.BlockSpec((sb, D), lambda i: (i, 0)),
            pl.BlockSpec((sb, D // 128, 128), lambda i: (i, 0, 0)),
            pl.BlockSpec((sb, D // 128, 128), lambda i: (i + S // sb, 0, 0)),
            pl.BlockSpec((sb, 1), lambda i: (i, 0)),
            pl.BlockSpec((sb, 1), lambda i: (i, 0)),
        ],
        out_specs=pl.BlockSpec((sb, D), lambda i: (i, 0)),
        out_shape=jax.ShapeDtypeStruct((S, D), f32),
    )(hs2d, yg, yg, w0, w1)


def kernel(hidden_states, position_ids, input_ln_w, q_w, k_w, v_w, o_w,
           q_ln_w, k_ln_w, post_ln_w, router_w, gate_w, up_w, down_w):
    h2d = hidden_states.reshape(S, D)
    pos = position_ids.reshape(S).astype(f32)
    inv = 1.0 / (THETA ** (jnp.arange(0, HD, 2, dtype=f32) / HD))
    ang = pos[:, None] * inv[None, :]
    cos_t = jnp.concatenate([jnp.cos(ang), jnp.cos(ang)], axis=1)
    sin_t = jnp.concatenate([jnp.sin(ang), jnp.sin(ang)], axis=1)

    q2d, k2d, v2d = _qkv(h2d, input_ln_w, q_w, k_w, v_w)
    kr2d = _krope(k2d, cos_t, sin_t, k_ln_w)
    o2d = _attention(q2d, kr2d, v2d, cos_t, sin_t, q_ln_w)

    rw_pad = jnp.pad(router_w, ((0, 0), (0, 128 - E))).astype(bf16)
    hs2d, xn3, logits = _oproj(o2d, o_w, h2d, post_ln_w, rw_pad)

    ej, wj, rank, cnt = _route_sort(logits)
    ppos, rows, bexp_l = _desc(cnt, ej, rank)

    rows_i = rows.reshape(P).astype(jnp.int32)
    ppos_i = ppos.reshape(NA).astype(jnp.int32)
    bexp = bexp_l.reshape(128)[:NB].astype(jnp.int32)
    w0 = wj.reshape(TOPK, S, 1)[0]
    w1 = wj.reshape(TOPK, S, 1)[1]

    nl = D // 128
    xdisp3 = _make_sc_gather(S, P, nl, f32)(xn3, rows_i)
    ydisp3 = _moe_grouped(xdisp3, gate_w, up_w, down_w, bexp)
    yg = _make_sc_gather(P, NA, nl, f32)(ydisp3, ppos_i)
    out = _combine(hs2d, yg, w0, w1)
    return out.reshape(B, S, D)


# speedup vs baseline: 1.2279x; 1.1137x over previous
import functools
import math

import jax
import jax.numpy as jnp
from jax import lax
from jax.experimental import pallas as pl
from jax.experimental.pallas import tpu as pltpu
from jax.experimental.pallas import tpu_sc as plsc

B, S, D = 1, 2048, 2048
H, KV, HD = 16, 4, 128
E, TOPK, FF = 8, 2, 768
EPS = 1e-6
THETA = 10000.0
NEG = -1e9

NA = S * TOPK
BLK = 128
NB = NA // BLK + E
P = NB * BLK

f32 = jnp.float32
bf16 = jnp.bfloat16


def _qkv_body(h_ref, lnw_ref, qw_ref, kw_ref, vw_ref, q_ref, k_ref, v_ref):
    x = h_ref[...]
    ms = jnp.mean(x * x, axis=1, keepdims=True)
    xn = x * jax.lax.rsqrt(ms + EPS) * lnw_ref[...]
    xb = xn.astype(bf16)
    q_ref[...] = jnp.dot(xb, qw_ref[...], preferred_element_type=f32)
    k_ref[...] = jnp.dot(xb, kw_ref[...], preferred_element_type=f32)
    v_ref[...] = jnp.dot(xb, vw_ref[...],
                         preferred_element_type=f32).astype(bf16)


def _qkv(h2d, input_ln_w, q_w, k_w, v_w, sb=256):
    n = S // sb
    return pl.pallas_call(
        _qkv_body,
        grid=(n,),
        in_specs=[
            pl.BlockSpec((sb, D), lambda i: (i, 0)),
            pl.BlockSpec((1, D), lambda i: (0, 0)),
            pl.BlockSpec((D, H * HD), lambda i: (0, 0)),
            pl.BlockSpec((D, KV * HD), lambda i: (0, 0)),
            pl.BlockSpec((D, KV * HD), lambda i: (0, 0)),
        ],
        out_specs=[
            pl.BlockSpec((sb, H * HD), lambda i: (i, 0)),
            pl.BlockSpec((sb, KV * HD), lambda i: (i, 0)),
            pl.BlockSpec((sb, KV * HD), lambda i: (i, 0)),
        ],
        out_shape=[
            jax.ShapeDtypeStruct((S, H * HD), f32),
            jax.ShapeDtypeStruct((S, KV * HD), f32),
            jax.ShapeDtypeStruct((S, KV * HD), bf16),
        ],
    )(h2d, input_ln_w.reshape(1, D), q_w.astype(bf16), k_w.astype(bf16),
      v_w.astype(bf16))


def _rot_cat(x):
    return jnp.concatenate([-x[:, HD // 2:], x[:, :HD // 2]], axis=1)


def _krope_body(k_ref, cos_ref, sin_ref, lnw_ref, o_ref):
    k = k_ref[...]
    ms = jnp.mean(k * k, axis=1, keepdims=True)
    kn = k * jax.lax.rsqrt(ms + EPS) * lnw_ref[...]
    o_ref[...] = (kn * cos_ref[...] + _rot_cat(kn) * sin_ref[...]).astype(bf16)


def _krope(k2d, cos_t, sin_t, k_ln_w, sb=512):
    n = S // sb
    return pl.pallas_call(
        _krope_body,
        grid=(KV, n),
        in_specs=[
            pl.BlockSpec((sb, HD), lambda kv, i: (i, kv)),
            pl.BlockSpec((sb, HD), lambda kv, i: (i, 0)),
            pl.BlockSpec((sb, HD), lambda kv, i: (i, 0)),
            pl.BlockSpec((1, HD), lambda kv, i: (0, 0)),
        ],
        out_specs=pl.BlockSpec((sb, HD), lambda kv, i: (i, kv)),
        out_shape=jax.ShapeDtypeStruct((S, KV * HD), bf16),
    )(k2d, cos_t, sin_t, k_ln_w.reshape(1, HD))


def _attn_body(q_ref, k_ref, v_ref, cos_ref, sin_ref, lnw_ref, o_ref,
               *, qb, kw, r0):
    i = pl.program_id(1)
    q = q_ref[...]
    ms = jnp.mean(q * q, axis=1, keepdims=True)
    qn = q * jax.lax.rsqrt(ms + EPS) * lnw_ref[...]
    qr = (qn * cos_ref[...] + _rot_cat(qn) * sin_ref[...]).astype(bf16)
    scores = jax.lax.dot_general(
        qr, k_ref[...], (((1,), (1,)), ((), ())),
        preferred_element_type=f32) * (1.0 / math.sqrt(HD))
    row = r0 + i * qb + jax.lax.broadcasted_iota(jnp.int32, (qb, kw), 0)
    col = jax.lax.broadcasted_iota(jnp.int32, (qb, kw), 1)
    scores = jnp.where(col <= row, scores, NEG)
    m = jnp.max(scores, axis=1, keepdims=True)
    p = jnp.exp(scores - m)
    attn = (p / jnp.sum(p, axis=1, keepdims=True)).astype(bf16)
    o_ref[...] = jnp.dot(attn, v_ref[...],
                         preferred_element_type=f32).astype(bf16)


def _attention_band(q2d, kr2d, v2d, cos_t, sin_t, q_ln_w, r0, nrows, kw,
                    qb=256):
    n = nrows // qb
    i0 = r0 // qb
    return pl.pallas_call(
        functools.partial(_attn_body, qb=qb, kw=kw, r0=r0),
        grid=(H, n),
        in_specs=[
            pl.BlockSpec((qb, HD), lambda h, i: (i + i0, h)),
            pl.BlockSpec((kw, HD), lambda h, i: (0, h // (H // KV))),
            pl.BlockSpec((kw, HD), lambda h, i: (0, h // (H // KV))),
            pl.BlockSpec((qb, HD), lambda h, i: (i + i0, 0)),
            pl.BlockSpec((qb, HD), lambda h, i: (i + i0, 0)),
            pl.BlockSpec((1, HD), lambda h, i: (0, 0)),
        ],
        out_specs=pl.BlockSpec((qb, HD), lambda h, i: (i, h)),
        out_shape=jax.ShapeDtypeStruct((nrows, H * HD), bf16),
    )(q2d, kr2d, v2d, cos_t, sin_t, q_ln_w.reshape(1, HD))


def _oproj_body(oa_ref, ob_ref, ow_ref, hid_ref, lnw_ref, rw_ref, hs_ref,
                xn_ref, lg_ref, *, nlo):
    o = jnp.where(pl.program_id(0) < nlo, oa_ref[...], ob_ref[...])
    att = jnp.dot(o, ow_ref[...], preferred_element_type=f32)
    hs = hid_ref[...] + att
    hs_ref[...] = hs
    ms = jnp.mean(hs * hs, axis=1, keepdims=True)
    xn = hs * jax.lax.rsqrt(ms + EPS) * lnw_ref[...]
    xn_ref[...] = xn.reshape(xn.shape[0], D // 128, 128)
    lg_ref[...] = jnp.dot(xn.astype(bf16), rw_ref[...],
                          preferred_element_type=f32)


def _oproj(oa2d, ob2d, o_w, hid2d, post_ln_w, rw_pad, sb=256):
    n = S // sb
    nlo = (S // 2) // sb
    return pl.pallas_call(
        functools.partial(_oproj_body, nlo=nlo),
        grid=(n,),
        in_specs=[
            pl.BlockSpec((sb, H * HD), lambda i: (jnp.minimum(i, 3), 0)),
            pl.BlockSpec((sb, H * HD), lambda i: (jnp.maximum(i - 4, 0), 0)),
            pl.BlockSpec((H * HD, D), lambda i: (0, 0)),
            pl.BlockSpec((sb, D), lambda i: (i, 0)),
            pl.BlockSpec((1, D), lambda i: (0, 0)),
            pl.BlockSpec((D, 128), lambda i: (0, 0)),
        ],
        out_specs=[
            pl.BlockSpec((sb, D), lambda i: (i, 0)),
            pl.BlockSpec((sb, D // 128, 128), lambda i: (i, 0, 0)),
            pl.BlockSpec((sb, 128), lambda i: (i, 0)),
        ],
        out_shape=[
            jax.ShapeDtypeStruct((S, D), f32),
            jax.ShapeDtypeStruct((S, D // 128, 128), f32),
            jax.ShapeDtypeStruct((S, 128), f32),
        ],
    )(oa2d, ob2d, o_w.astype(bf16), hid2d, post_ln_w.reshape(1, D), rw_pad)


AB = S


def _route_body(lg_ref, ej_ref, wj_ref, rank_ref, cnt_ref, cnt):
    j = pl.program_id(0)

    @pl.when(j == 0)
    def _():
        cnt[...] = jnp.zeros_like(cnt)

    l = lg_ref[...]
    lane = jax.lax.broadcasted_iota(jnp.int32, (AB, 128), 1)
    valid = lane < E
    l = jnp.where(valid, l, -1e30)
    m0 = jnp.max(l, axis=1, keepdims=True)
    i0 = jnp.min(jnp.where(l >= m0, lane, 1000), axis=1, keepdims=True)
    l1 = jnp.where(lane == i0, -1e30, l)
    m1 = jnp.max(l1, axis=1, keepdims=True)
    i1 = jnp.min(jnp.where(l1 >= m1, lane, 1000), axis=1, keepdims=True)
    w0 = 1.0 / (1.0 + jnp.exp(m1 - m0))
    ej = jnp.where(j == 0, i0, i1)
    wj = jnp.where(j == 0, w0, 1.0 - w0)
    onehot = (lane == ej).astype(f32)
    rowi = jax.lax.broadcasted_iota(jnp.int32, (AB, AB), 0)
    coli = jax.lax.broadcasted_iota(jnp.int32, (AB, AB), 1)
    ltri = (coli < rowi).astype(bf16)
    pref = jnp.dot(ltri, onehot.astype(bf16), preferred_element_type=f32)
    rank = jnp.sum(onehot * (pref + cnt[...]), axis=1, keepdims=True)
    ej_ref[...] = ej.astype(f32)
    wj_ref[...] = wj
    rank_ref[...] = rank
    cnt[...] += jnp.sum(onehot, axis=0, keepdims=True)
    cnt_ref[...] = cnt[...]


def _route_sort(logits):
    return pl.pallas_call(
        _route_body,
        grid=(TOPK,),
        in_specs=[pl.BlockSpec((AB, 128), lambda j: (0, 0))],
        out_specs=[
            pl.BlockSpec((AB, 1), lambda j: (j, 0)),
            pl.BlockSpec((AB, 1), lambda j: (j, 0)),
            pl.BlockSpec((AB, 1), lambda j: (j, 0)),
            pl.BlockSpec((1, 128), lambda j: (0, 0)),
        ],
        out_shape=[
            jax.ShapeDtypeStruct((NA, 1), f32),
            jax.ShapeDtypeStruct((NA, 1), f32),
            jax.ShapeDtypeStruct((NA, 1), f32),
            jax.ShapeDtypeStruct((1, 128), f32),
        ],
        scratch_shapes=[pltpu.VMEM((1, 128), f32)],
    )(logits)


def _desc_body(cnt_ref, ej_ref, rank_ref, ppos_ref, rows_ref, bexp_ref):
    cnts = [cnt_ref[0, e] for e in range(E)]
    pcs = [jnp.ceil(c / BLK) * BLK for c in cnts]
    pb = [f32(0.0)]
    ab = [f32(0.0)]
    for e in range(E):
        pb.append(pb[e] + pcs[e])
        ab.append(ab[e] + cnts[e])
    ej = ej_ref[...]
    acc = jnp.zeros_like(ej)
    for e in range(E):
        acc += jnp.where(ej == e, pb[e], 0.0)
    ppos = acc + rank_ref[...]
    ppos_ref[...] = ppos
    hi = jnp.floor(ppos * (1.0 / 128.0))
    lo = ppos - hi * 128.0
    mhi = (hi == jax.lax.broadcasted_iota(jnp.int32, (NA, NB), 1)
           .astype(f32)).astype(f32)
    a0 = jax.lax.broadcasted_iota(jnp.int32, (NA, 1), 0)
    tok = (a0 % S).astype(f32)
    mlo = jnp.where(
        lo == jax.lax.broadcasted_iota(jnp.int32, (NA, 128), 1).astype(f32),
        tok, 0.0)
    rows_ref[...] = jax.lax.dot_general(
        mhi, mlo, (((0,), (0,)), ((), ())),
        precision=jax.lax.Precision.HIGHEST,
        preferred_element_type=f32)
    lb = jax.lax.broadcasted_iota(jnp.int32, (1, 128), 1).astype(f32) * BLK
    bacc = -jnp.ones((1, 128), f32)
    for e in range(E):
        bacc += jnp.where(lb >= pb[e], 1.0, 0.0)
    bexp_ref[...] = bacc


def _desc(cnt, ej, rank):
    return pl.pallas_call(
        _desc_body,
        grid=(1,),
        in_specs=[
            pl.BlockSpec(memory_space=pltpu.SMEM),
            pl.BlockSpec((NA, 1), lambda i: (0, 0)),
            pl.BlockSpec((NA, 1), lambda i: (0, 0)),
        ],
        out_specs=[
            pl.BlockSpec((NA, 1), lambda i: (0, 0)),
            pl.BlockSpec((NB, 128), lambda i: (0, 0)),
            pl.BlockSpec((1, 128), lambda i: (0, 0)),
        ],
        out_shape=[
            jax.ShapeDtypeStruct((NA, 1), f32),
            jax.ShapeDtypeStruct((NB, 128), f32),
            jax.ShapeDtypeStruct((1, 128), f32),
        ],
    )(cnt, ej, rank)


def _make_sc_gather(nrows_tab, nrows_out, sl, dtype):
    info = plsc.get_sparse_core_info()
    nw = info.num_cores * info.num_subcores
    n_per_w = nrows_out // nw
    row_bytes = sl * 128 * jnp.dtype(dtype).itemsize
    ch = 8
    cand = 8
    while cand <= n_per_w:
        if n_per_w % cand == 0 and 2 * cand * row_bytes <= 440_000:
            ch = cand
        cand += 8
    nch = n_per_w // ch
    mesh = plsc.VectorSubcoreMesh(core_axis_name="c", subcore_axis_name="s")

    @functools.partial(
        pl.kernel, mesh=mesh,
        out_type=jax.ShapeDtypeStruct((nrows_out, sl, 128), dtype),
        scratch_types=[
            pltpu.VMEM((2, ch), jnp.int32),
            pltpu.VMEM((ch, sl, 128), dtype),
            pltpu.VMEM((ch, sl, 128), dtype),
            pltpu.SemaphoreType.DMA,
            pltpu.SemaphoreType.DMA,
            pltpu.SemaphoreType.DMA,
            pltpu.SemaphoreType.DMA,
        ],
    )
    def k(tab_hbm, idx_hbm, out_hbm, idx_v, buf0, buf1, g0, g1, w0, w1):
        wid = lax.axis_index("s") * info.num_cores + lax.axis_index("c")
        base = wid * n_per_w
        bufs = (buf0, buf1)
        gs = (g0, g1)
        ws = (w0, w1)

        def start_gather(c):
            b = c & 1
            pltpu.sync_copy(idx_hbm.at[pl.ds(base + c * ch, ch)],
                            idx_v.at[b])
            return pltpu.async_copy(tab_hbm.at[idx_v.at[b]], bufs[b], gs[b])

        gh = {0: start_gather(0)}
        wh = {}
        for c in range(nch):
            b = c & 1
            gh[c].wait()
            if c + 1 < nch:
                if c >= 1:
                    wh[c - 1].wait()
                gh[c + 1] = start_gather(c + 1)
            wh[c] = pltpu.async_copy(
                bufs[b], out_hbm.at[pl.ds(base + c * ch, ch)], ws[b])
        if nch >= 2:
            wh[nch - 2].wait()
        wh[nch - 1].wait()

    return k


def _moe_body(be_ref, x_ref, gw_ref, uw_ref, dw_ref, y_ref, gwb, uwb, dwb):
    b = pl.program_id(0)
    prev = be_ref[jnp.maximum(b - 1, 0)]

    @pl.when((b == 0) | (be_ref[b] != prev))
    def _():
        gwb[...] = gw_ref[0].astype(bf16)
        uwb[...] = uw_ref[0].astype(bf16)
        dwb[...] = dw_ref[0].astype(bf16)

    x = x_ref[...].reshape(BLK, D).astype(bf16)
    g = jnp.dot(x, gwb[...], preferred_element_type=f32)
    u = jnp.dot(x, uwb[...], preferred_element_type=f32)
    a = ((g / (1.0 + jnp.exp(-g))) * u).astype(bf16)
    y = jnp.dot(a, dwb[...], preferred_element_type=f32)
    y_ref[...] = y.reshape(BLK, D // 128, 128)


def _moe_grouped(xdisp3, gate_w, up_w, down_w, bexp):
    grid_spec = pltpu.PrefetchScalarGridSpec(
        num_scalar_prefetch=1,
        grid=(NB,),
        in_specs=[
            pl.BlockSpec((BLK, D // 128, 128), lambda b, be: (b, 0, 0)),
            pl.BlockSpec((1, D, FF), lambda b, be: (be[b], 0, 0)),
            pl.BlockSpec((1, D, FF), lambda b, be: (be[b], 0, 0)),
            pl.BlockSpec((1, FF, D), lambda b, be: (be[b], 0, 0)),
        ],
        out_specs=pl.BlockSpec((BLK, D // 128, 128), lambda b, be: (b, 0, 0)),
        scratch_shapes=[
            pltpu.VMEM((D, FF), bf16),
            pltpu.VMEM((D, FF), bf16),
            pltpu.VMEM((FF, D), bf16),
        ],
    )
    return pl.pallas_call(
        _moe_body,
        grid_spec=grid_spec,
        out_shape=jax.ShapeDtypeStruct((P, D // 128, 128), f32),
    )(bexp, xdisp3, gate_w, up_w, down_w)


def _comb_body(hs_ref, y0_ref, y1_ref, w0_ref, w1_ref, out_ref):
    sb = hs_ref.shape[0]
    y0 = y0_ref[...].reshape(sb, D)
    y1 = y1_ref[...].reshape(sb, D)
    out_ref[...] = (hs_ref[...] + w0_ref[...] * y0 + w1_ref[...] * y1)


def _combine(hs2d, yg, w0, w1, sb=256):
    n = S // sb
    return pl.pallas_call(
        _comb_body,
        grid=(n,),
        in_specs=[
            pl.BlockSpec((sb, D), lambda i: (i, 0)),
            pl.BlockSpec((sb, D // 128, 128), lambda i: (i, 0, 0)),
            pl.BlockSpec((sb, D // 128, 128), lambda i: (i + S // sb, 0, 0)),
            pl.BlockSpec((sb, 1), lambda i: (i, 0)),
            pl.BlockSpec((sb, 1), lambda i: (i, 0)),
        ],
        out_specs=pl.BlockSpec((sb, D), lambda i: (i, 0)),
        out_shape=jax.ShapeDtypeStruct((S, D), f32),
    )(hs2d, yg, yg, w0, w1)


def kernel(hidden_states, position_ids, input_ln_w, q_w, k_w, v_w, o_w,
           q_ln_w, k_ln_w, post_ln_w, router_w, gate_w, up_w, down_w):
    h2d = hidden_states.reshape(S, D)
    pos = position_ids.reshape(S).astype(f32)
    inv = 1.0 / (THETA ** (jnp.arange(0, HD, 2, dtype=f32) / HD))
    ang = pos[:, None] * inv[None, :]
    cos_t = jnp.concatenate([jnp.cos(ang), jnp.cos(ang)], axis=1)
    sin_t = jnp.concatenate([jnp.sin(ang), jnp.sin(ang)], axis=1)

    q2d, k2d, v2d = _qkv(h2d, input_ln_w, q_w, k_w, v_w)
    kr2d = _krope(k2d, cos_t, sin_t, k_ln_w)
    oa2d = _attention_band(q2d, kr2d, v2d, cos_t, sin_t, q_ln_w,
                           0, S // 2, S // 2)
    ob2d = _attention_band(q2d, kr2d, v2d, cos_t, sin_t, q_ln_w,
                           S // 2, S // 2, S)

    rw_pad = jnp.pad(router_w, ((0, 0), (0, 128 - E))).astype(bf16)
    hs2d, xn3, logits = _oproj(oa2d, ob2d, o_w, h2d, post_ln_w, rw_pad)

    ej, wj, rank, cnt = _route_sort(logits)
    ppos, rows, bexp_l = _desc(cnt, ej, rank)

    rows_i = rows.reshape(P).astype(jnp.int32)
    ppos_i = ppos.reshape(NA).astype(jnp.int32)
    bexp = bexp_l.reshape(128)[:NB].astype(jnp.int32)
    w0 = wj.reshape(TOPK, S, 1)[0]
    w1 = wj.reshape(TOPK, S, 1)[1]

    nl = D // 128
    xdisp3 = _make_sc_gather(S, P, nl, f32)(xn3, rows_i)
    ydisp3 = _moe_grouped(xdisp3, gate_w, up_w, down_w, bexp)
    yg = _make_sc_gather(P, NA, nl, f32)(ydisp3, ppos_i)
    out = _combine(hs2d, yg, w0, w1)
    return out.reshape(B, S, D)
